# baseline (speedup 1.0000x reference)
import functools

import jax
import jax.numpy as jnp
from jax import lax
from jax.experimental import pallas as pl
from jax.experimental.pallas import tpu as pltpu
from jax.experimental.pallas import tpu_sc as plsc

NC = 2
NS = 16
CH = 128
NB = 2
NQ = 4


def _elu(x):
    return jnp.where(x > 0, x, jnp.exp(jnp.minimum(x, 0.0)) - 1.0)


def _mm_body(x_ref, w_ref, o_ref):
    o_ref[...] = jnp.dot(x_ref[...], w_ref[...],
                         preferred_element_type=jnp.float32)


def _tc_matmul(x, w, bm):
    n, k = x.shape
    m = w.shape[1]
    return pl.pallas_call(
        _mm_body,
        grid=(n // bm,),
        in_specs=[pl.BlockSpec((bm, k), lambda i: (i, 0)),
                  pl.BlockSpec((k, m), lambda i: (0, 0))],
        out_specs=pl.BlockSpec((bm, m), lambda i: (i, 0)),
        out_shape=jax.ShapeDtypeStruct((n, m), jnp.float32),
    )(x, w)


def _elu_mm_body(p_ref, w_ref, o_ref):
    a = p_ref[0] + p_ref[1]
    o_ref[...] = jnp.dot(_elu(a), w_ref[...],
                         preferred_element_type=jnp.float32)


def _tc_elu_matmul(p, w, n, bm):
    k = p.shape[2]
    m = w.shape[1]
    return pl.pallas_call(
        _elu_mm_body,
        grid=(n // bm,),
        in_specs=[pl.BlockSpec((2, bm, k), lambda i: (0, i, 0)),
                  pl.BlockSpec((k, m), lambda i: (0, 0))],
        out_specs=pl.BlockSpec((bm, m), lambda i: (i, 0)),
        out_shape=jax.ShapeDtypeStruct((n, m), jnp.float32),
    )(p, w)


def _head_body(p_ref, wb_ref, h_ref, q_ref):
    d = wb_ref.shape[0]
    h = _elu(p_ref[0, :, :d] + p_ref[1, :, :d])
    ws = (wb_ref[...] + wb_ref[...].T) * 0.5
    h_ref[...] = h
    q_ref[...] = jnp.dot(h, ws, preferred_element_type=jnp.float32)


def _tc_head(p, wb, n, bm):
    dp = p.shape[2]
    d = wb.shape[0]
    return pl.pallas_call(
        _head_body,
        grid=(n // bm,),
        in_specs=[pl.BlockSpec((2, bm, dp), lambda i: (0, i, 0)),
                  pl.BlockSpec((d, d), lambda i: (0, 0))],
        out_specs=[pl.BlockSpec((bm, d), lambda i: (i, 0)),
                   pl.BlockSpec((bm, d), lambda i: (i, 0))],
        out_shape=[jax.ShapeDtypeStruct((n, d), jnp.float32),
                   jax.ShapeDtypeStruct((n, d), jnp.float32)],
    )(p, wb)


def _bilin_body(q_ref, h_ref, o_ref):
    t = lax.dot_general(q_ref[...], h_ref[...], (((1,), (1,)), ((), ())),
                        preferred_element_type=jnp.float32)
    o_ref[...] = 1.0 / (1.0 + jnp.exp(-t))


def _tc_bilinear(q, h, bmi, bmj):
    n, d = q.shape
    return pl.pallas_call(
        _bilin_body,
        grid=(pl.cdiv(n, bmi), pl.cdiv(n, bmj)),
        in_specs=[pl.BlockSpec((bmi, d), lambda i, j: (i, 0)),
                  pl.BlockSpec((bmj, d), lambda i, j: (j, 0))],
        out_specs=pl.BlockSpec((bmi, bmj), lambda i, j: (i, j)),
        out_shape=jax.ShapeDtypeStruct((n, n), jnp.float32),
    )(q, h)


def _bcast_lane(v16, e):
    idx = jnp.full((16, 1), e, jnp.int32)
    dn = lax.GatherDimensionNumbers(
        offset_dims=(), collapsed_slice_dims=(0,), start_index_map=(0,))
    return lax.gather(v16, idx, dn, (1,),
                      mode=lax.GatherScatterMode.PROMISE_IN_BOUNDS)


def _make_spmm(np_, d, e_pad, tc_tiling=True):
    per_core = e_pad // NC
    per_tile = per_core // NS
    n_chunks = per_tile // CH
    assert n_chunks % (NB * NQ // 2) == 0
    rpt = np_ // NS
    zr = CH
    lanes = d // 16
    mesh = plsc.VectorSubcoreMesh(core_axis_name="c", subcore_axis_name="s")

    @functools.partial(
        pl.kernel,
        mesh=mesh,
        out_type=jax.ShapeDtypeStruct((NC, np_, d), jnp.float32),
        compiler_params=pltpu.CompilerParams(use_tc_tiling_on_sc=tc_tiling),
        scratch_types=[
            pltpu.VMEM((NQ, CH), jnp.int32),
            pltpu.VMEM((NQ * CH // 16, 16), jnp.int32),
            pltpu.VMEM((NQ, CH), jnp.float32),
            [pltpu.VMEM((CH, d), jnp.float32) for _ in range(NB)],
            pltpu.VMEM_SHARED((np_, d), jnp.float32),
            [pltpu.SemaphoreType.DMA for _ in range(NB)],
            [pltpu.SemaphoreType.DMA for _ in range(NQ)],
            [pltpu.SemaphoreType.DMA for _ in range(NB)],
            pltpu.SemaphoreType.DMA,
        ],
    )
    def spmm(h_hbm, cols_hbm, rows_hbm, vals_hbm, out_hbm,
             colv, rowv, valv, gath, acc, gsem, isem, ssem, sem):
        c = lax.axis_index("c")
        s = lax.axis_index("s")

        zeros16 = jnp.zeros((16,), jnp.float32)

        @pl.loop(0, zr)
        def _zero_gath(r):
            for j in range(lanes):
                gath[0][r, pl.ds(j * 16, 16)] = zeros16

        stripe = s * rpt
        zcopies = [
            pltpu.make_async_copy(
                gath[0], acc.at[pl.ds(stripe + t * zr, zr)], sem)
            for t in range(rpt // zr)
        ]
        for cp in zcopies:
            cp.start()
        for cp in zcopies:
            cp.wait()
        plsc.subcore_barrier()

        cb = (c * NS + s) * n_chunks

        ng = CH // 16

        def idx_copies(k, q):
            ci = cb + k
            return [
                pltpu.make_async_copy(cols_hbm.at[ci], colv.at[q], isem[q]),
                pltpu.make_async_copy(rows_hbm.at[pl.ds(ci * ng, ng)],
                                      rowv.at[pl.ds(q * ng, ng)], isem[q]),
                pltpu.make_async_copy(vals_hbm.at[ci], valv.at[q], isem[q]),
            ]

        def fire_idx(k, q):
            for cp in idx_copies(k, q):
                cp.start()

        def drain_idx(k, q):
            for cp in idx_copies(k, q):
                cp.wait()

        def start_gather(q, b):
            pltpu.async_copy(h_hbm.at[colv.at[q]], gath[b], gsem[b])

        def wait_gather(q, b):
            pltpu.make_async_copy(h_hbm.at[colv.at[q]], gath[b], gsem[b]).wait()

        fire_idx(0, 0)
        fire_idx(1, 1)
        drain_idx(0, 0)
        start_gather(0, 0)
        drain_idx(1, 1)
        start_gather(1, 1)

        @pl.loop(0, n_chunks, step=NQ)
        def _chunk(k):
            for q in range(NQ):
                kk = k + q
                b = q % NB
                qn = (q + 2) % NQ

                @pl.when(kk + 2 < n_chunks)
                def _fire_next_idx():
                    fire_idx(kk + 2, qn)

                wait_gather(q, b)
                g = gath[b]

                def grp_scatter(gi):
                    return pltpu.make_async_copy(
                        g.at[pl.ds(gi * 16, 16)],
                        acc.at[rowv.at[q * ng + gi]], ssem[b])

                @pl.loop(0, ng)
                def _scale(gi):
                    ebase = gi * 16
                    v16 = valv[q, pl.ds(ebase, 16)]
                    for e in range(16):
                        vv = _bcast_lane(v16, e)
                        for j in range(lanes):
                            sl = pl.ds(j * 16, 16)
                            g[ebase + e, sl] = g[ebase + e, sl] * vv
                    grp_scatter(gi).start(add=True)

                @pl.loop(0, ng)
                def _drain(gi):
                    grp_scatter(gi).wait()

                @pl.when(kk + 2 < n_chunks)
                def _issue_next():
                    drain_idx(kk + 2, qn)
                    start_gather(qn, b)

        plsc.subcore_barrier()
        ocopies = [
            pltpu.make_async_copy(
                acc.at[pl.ds(stripe + t * zr, zr)],
                out_hbm.at[c, pl.ds(stripe + t * zr, zr)], sem)
            for t in range(rpt // zr)
        ]
        for cp in ocopies:
            cp.start()
        for cp in ocopies:
            cp.wait()

    return spmm


def _spmm_partials(h, cols, rows, vals):
    d = h.shape[1]
    np_ = ((h.shape[0] + NS * 128 - 1) // (NS * 128)) * (NS * 128)
    e_pad = cols.shape[0]
    shp = (e_pad // CH, CH)
    return _make_spmm(np_, d, e_pad, tc_tiling=(d % 128 == 0))(
        h, cols.reshape(shp), rows.reshape(-1, 16), vals.reshape(shp))


def kernel(x, adj_indices, adj_values, W1, W2, Wb):
    n = x.shape[0]
    e = adj_values.shape[0]
    group = NC * NS * CH * NB
    e_pad = ((e + group - 1) // group) * group

    rows = jnp.concatenate(
        [adj_indices[0], jnp.zeros((e_pad - e,), jnp.int32)])
    cols = jnp.concatenate(
        [adj_indices[1], jnp.zeros((e_pad - e,), jnp.int32)])
    vals = jnp.concatenate(
        [adj_values, jnp.zeros((e_pad - e,), jnp.float32)])

    g1 = _tc_matmul(x, W1, bm=2000)
    p1 = _spmm_partials(g1, cols, rows, vals)
    g2 = _tc_elu_matmul(p1, W2, n, bm=2000)
    p2 = _spmm_partials(g2, cols, rows, vals)
    h2, q = _tc_head(p2, Wb, n, bm=2000)
    return _tc_bilinear(q, h2, bmi=2048, bmj=2048)


# speedup vs baseline: 1.0423x; 1.0423x over previous
import functools

import jax
import jax.numpy as jnp
from jax import lax
from jax.experimental import pallas as pl
from jax.experimental.pallas import tpu as pltpu
from jax.experimental.pallas import tpu_sc as plsc

NC = 2
NS = 16
CH = 128
NB = 2
NQ = 4


def _elu(x):
    return jnp.where(x > 0, x, jnp.exp(jnp.minimum(x, 0.0)) - 1.0)


def _mm_body(x_ref, w_ref, o_ref):
    o_ref[...] = jnp.dot(x_ref[...], w_ref[...],
                         preferred_element_type=jnp.float32)


def _tc_matmul(x, w, bm):
    n, k = x.shape
    m = w.shape[1]
    return pl.pallas_call(
        _mm_body,
        grid=(n // bm,),
        in_specs=[pl.BlockSpec((bm, k), lambda i: (i, 0)),
                  pl.BlockSpec((k, m), lambda i: (0, 0))],
        out_specs=pl.BlockSpec((bm, m), lambda i: (i, 0)),
        out_shape=jax.ShapeDtypeStruct((n, m), jnp.float32),
    )(x, w)


def _elu_mm_body(p_ref, w_ref, o_ref):
    a = p_ref[0] + p_ref[1]
    o_ref[...] = jnp.dot(_elu(a), w_ref[...],
                         preferred_element_type=jnp.float32)


def _tc_elu_matmul(p, w, n, bm):
    k = p.shape[2]
    m = w.shape[1]
    return pl.pallas_call(
        _elu_mm_body,
        grid=(n // bm,),
        in_specs=[pl.BlockSpec((2, bm, k), lambda i: (0, i, 0)),
                  pl.BlockSpec((k, m), lambda i: (0, 0))],
        out_specs=pl.BlockSpec((bm, m), lambda i: (i, 0)),
        out_shape=jax.ShapeDtypeStruct((n, m), jnp.float32),
    )(p, w)


def _head_body(p_ref, wb_ref, h_ref, q_ref):
    d = wb_ref.shape[0]
    h = _elu(p_ref[0, :, :d] + p_ref[1, :, :d])
    ws = (wb_ref[...] + wb_ref[...].T) * 0.5
    h_ref[...] = h
    q_ref[...] = jnp.dot(h, ws, preferred_element_type=jnp.float32)


def _tc_head(p, wb, n, bm):
    dp = p.shape[2]
    d = wb.shape[0]
    return pl.pallas_call(
        _head_body,
        grid=(n // bm,),
        in_specs=[pl.BlockSpec((2, bm, dp), lambda i: (0, i, 0)),
                  pl.BlockSpec((d, d), lambda i: (0, 0))],
        out_specs=[pl.BlockSpec((bm, d), lambda i: (i, 0)),
                   pl.BlockSpec((bm, d), lambda i: (i, 0))],
        out_shape=[jax.ShapeDtypeStruct((n, d), jnp.float32),
                   jax.ShapeDtypeStruct((n, d), jnp.float32)],
    )(p, wb)


def _bilin_body(q_ref, h_ref, o_ref):
    t = lax.dot_general(q_ref[...], h_ref[...], (((1,), (1,)), ((), ())),
                        preferred_element_type=jnp.float32)
    o_ref[...] = 1.0 / (1.0 + jnp.exp(-t))


def _tc_bilinear(q, h, bmi, bmj):
    n, d = q.shape
    return pl.pallas_call(
        _bilin_body,
        grid=(pl.cdiv(n, bmi), pl.cdiv(n, bmj)),
        in_specs=[pl.BlockSpec((bmi, d), lambda i, j: (i, 0)),
                  pl.BlockSpec((bmj, d), lambda i, j: (j, 0))],
        out_specs=pl.BlockSpec((bmi, bmj), lambda i, j: (i, j)),
        out_shape=jax.ShapeDtypeStruct((n, n), jnp.float32),
    )(q, h)


def _bcast_lane(v16, e):
    idx = jnp.full((16, 1), e, jnp.int32)
    dn = lax.GatherDimensionNumbers(
        offset_dims=(), collapsed_slice_dims=(0,), start_index_map=(0,))
    return lax.gather(v16, idx, dn, (1,),
                      mode=lax.GatherScatterMode.PROMISE_IN_BOUNDS)


def _make_spmm(np_, d, e_pad, tc_tiling=True):
    per_core = e_pad // NC
    per_tile = per_core // NS
    n_chunks = per_tile // CH
    assert n_chunks % (NB * NQ // 2) == 0
    rpt = np_ // NS
    zr = CH
    lanes = d // 16
    mesh = plsc.VectorSubcoreMesh(core_axis_name="c", subcore_axis_name="s")

    @functools.partial(
        pl.kernel,
        mesh=mesh,
        out_type=jax.ShapeDtypeStruct((NC, np_, d), jnp.float32),
        compiler_params=pltpu.CompilerParams(use_tc_tiling_on_sc=tc_tiling),
        scratch_types=[
            pltpu.VMEM((NQ, CH), jnp.int32),
            pltpu.VMEM((NQ * CH // 16, 16), jnp.int32),
            pltpu.VMEM((NQ, CH), jnp.float32),
            [pltpu.VMEM((CH, d), jnp.float32) for _ in range(NB)],
            pltpu.VMEM_SHARED((np_, d), jnp.float32),
            [pltpu.SemaphoreType.DMA for _ in range(NB)],
            [pltpu.SemaphoreType.DMA for _ in range(NQ)],
            [pltpu.SemaphoreType.DMA for _ in range(NB)],
            pltpu.SemaphoreType.DMA,
        ],
    )
    def spmm(h_hbm, cols_hbm, rows_hbm, vals_hbm, out_hbm,
             colv, rowv, valv, gath, acc, gsem, isem, ssem, sem):
        c = lax.axis_index("c")
        s = lax.axis_index("s")

        zeros16 = jnp.zeros((16,), jnp.float32)

        @pl.loop(0, zr)
        def _zero_gath(r):
            for j in range(lanes):
                gath[0][r, pl.ds(j * 16, 16)] = zeros16

        stripe = s * rpt
        zcopies = [
            pltpu.make_async_copy(
                gath[0], acc.at[pl.ds(stripe + t * zr, zr)], sem)
            for t in range(rpt // zr)
        ]
        for cp in zcopies:
            cp.start()
        for cp in zcopies:
            cp.wait()
        plsc.subcore_barrier()

        cb = (c * NS + s) * n_chunks

        ng = CH // 16

        def idx_copies(k, q):
            ci = cb + k
            return [
                pltpu.make_async_copy(cols_hbm.at[ci], colv.at[q], isem[q]),
                pltpu.make_async_copy(rows_hbm.at[pl.ds(ci * ng, ng)],
                                      rowv.at[pl.ds(q * ng, ng)], isem[q]),
                pltpu.make_async_copy(vals_hbm.at[ci], valv.at[q], isem[q]),
            ]

        def fire_idx(k, q):
            for cp in idx_copies(k, q):
                cp.start()

        def drain_idx(k, q):
            for cp in idx_copies(k, q):
                cp.wait()

        def start_gather(q, b):
            pltpu.async_copy(h_hbm.at[colv.at[q]], gath[b], gsem[b])

        def wait_gather(q, b):
            pltpu.make_async_copy(h_hbm.at[colv.at[q]], gath[b], gsem[b]).wait()

        fire_idx(0, 0)
        fire_idx(1, 1)
        drain_idx(0, 0)
        start_gather(0, 0)
        drain_idx(1, 1)
        start_gather(1, 1)

        @pl.loop(0, n_chunks, step=NQ)
        def _chunk(k):
            for q in range(NQ):
                kk = k + q
                b = q % NB
                qn = (q + 2) % NQ

                @pl.when(kk + 2 < n_chunks)
                def _fire_next_idx():
                    fire_idx(kk + 2, qn)

                wait_gather(q, b)
                g = gath[b]

                def grp_scatter(gi):
                    return pltpu.make_async_copy(
                        g.at[pl.ds(gi * 16, 16)],
                        acc.at[rowv.at[q * ng + gi]], ssem[b])

                @pl.loop(0, ng)
                def _scale(gi):
                    ebase = gi * 16
                    v16 = valv[q, pl.ds(ebase, 16)]
                    for e in range(16):
                        vv = _bcast_lane(v16, e)
                        for j in range(lanes):
                            sl = pl.ds(j * 16, 16)
                            g[ebase + e, sl] = g[ebase + e, sl] * vv
                    grp_scatter(gi).start(add=True)

                @pl.loop(0, ng)
                def _drain(gi):
                    grp_scatter(gi).wait()

                @pl.when(kk + 2 < n_chunks)
                def _issue_next():
                    drain_idx(kk + 2, qn)
                    start_gather(qn, b)

        plsc.subcore_barrier()
        ocopies = [
            pltpu.make_async_copy(
                acc.at[pl.ds(stripe + t * zr, zr)],
                out_hbm.at[c, pl.ds(stripe + t * zr, zr)], sem)
            for t in range(rpt // zr)
        ]
        for cp in ocopies:
            cp.start()
        for cp in ocopies:
            cp.wait()

    return spmm


def _spmm_partials(h, cols, rows, vals):
    d = h.shape[1]
    np_ = ((h.shape[0] + NS * 128 - 1) // (NS * 128)) * (NS * 128)
    e_pad = cols.shape[0]
    shp = (e_pad // CH, CH)
    return _make_spmm(np_, d, e_pad)(
        h, cols.reshape(shp), rows.reshape(-1, 16), vals.reshape(shp))


def kernel(x, adj_indices, adj_values, W1, W2, Wb):
    n = x.shape[0]
    e = adj_values.shape[0]
    group = NC * NS * CH * NB
    e_pad = ((e + group - 1) // group) * group

    rows = jnp.concatenate(
        [adj_indices[0], jnp.zeros((e_pad - e,), jnp.int32)])
    cols = jnp.concatenate(
        [adj_indices[1], jnp.zeros((e_pad - e,), jnp.int32)])
    vals = jnp.concatenate(
        [adj_values, jnp.zeros((e_pad - e,), jnp.float32)])

    w2p = jnp.pad(W2, ((0, 0), (0, 128 - W2.shape[1])))

    g1 = _tc_matmul(x, W1, bm=2000)
    p1 = _spmm_partials(g1, cols, rows, vals)
    g2 = _tc_elu_matmul(p1, w2p, n, bm=2000)
    p2 = _spmm_partials(g2, cols, rows, vals)
    h2, q = _tc_head(p2, Wb, n, bm=2000)
    return _tc_bilinear(q, h2, bmi=2048, bmj=2048)


# speedup vs baseline: 1.0627x; 1.0196x over previous
import functools

import jax
import jax.numpy as jnp
from jax import lax
from jax.experimental import pallas as pl
from jax.experimental.pallas import tpu as pltpu
from jax.experimental.pallas import tpu_sc as plsc

NC = 2
NS = 16
CH = 128
NB = 2
NQ = 4


def _elu(x):
    return jnp.where(x > 0, x, jnp.exp(jnp.minimum(x, 0.0)) - 1.0)


def _mm_body(x_ref, w_ref, o_ref):
    o_ref[...] = jnp.dot(x_ref[...], w_ref[...],
                         preferred_element_type=jnp.float32)


def _tc_matmul(x, w, bm):
    n, k = x.shape
    m = w.shape[1]
    return pl.pallas_call(
        _mm_body,
        grid=(n // bm,),
        in_specs=[pl.BlockSpec((bm, k), lambda i: (i, 0)),
                  pl.BlockSpec((k, m), lambda i: (0, 0))],
        out_specs=pl.BlockSpec((bm, m), lambda i: (i, 0)),
        out_shape=jax.ShapeDtypeStruct((n, m), jnp.float32),
    )(x, w)


def _elu_mm_body(p_ref, w_ref, o_ref):
    a = p_ref[0] + p_ref[1]
    o_ref[...] = jnp.dot(_elu(a), w_ref[...],
                         preferred_element_type=jnp.float32)


def _tc_elu_matmul(p, w, n, bm):
    k = p.shape[2]
    m = w.shape[1]
    return pl.pallas_call(
        _elu_mm_body,
        grid=(n // bm,),
        in_specs=[pl.BlockSpec((2, bm, k), lambda i: (0, i, 0)),
                  pl.BlockSpec((k, m), lambda i: (0, 0))],
        out_specs=pl.BlockSpec((bm, m), lambda i: (i, 0)),
        out_shape=jax.ShapeDtypeStruct((n, m), jnp.float32),
    )(p, w)


def _head_body(p_ref, wb_ref, h_ref, q_ref):
    d = wb_ref.shape[0]
    h = _elu(p_ref[0, :, :d] + p_ref[1, :, :d])
    ws = (wb_ref[...] + wb_ref[...].T) * 0.25
    h_ref[...] = h
    q_ref[...] = jnp.dot(h, ws, preferred_element_type=jnp.float32)


def _tc_head(p, wb, n, bm):
    dp = p.shape[2]
    d = wb.shape[0]
    return pl.pallas_call(
        _head_body,
        grid=(n // bm,),
        in_specs=[pl.BlockSpec((2, bm, dp), lambda i: (0, i, 0)),
                  pl.BlockSpec((d, d), lambda i: (0, 0))],
        out_specs=[pl.BlockSpec((bm, d), lambda i: (i, 0)),
                   pl.BlockSpec((bm, d), lambda i: (i, 0))],
        out_shape=[jax.ShapeDtypeStruct((n, d), jnp.float32),
                   jax.ShapeDtypeStruct((n, d), jnp.float32)],
    )(p, wb)


def _bilin_body(q_ref, h_ref, o_ref):
    t = lax.dot_general(q_ref[...], h_ref[...], (((1,), (1,)), ((), ())),
                        preferred_element_type=jnp.float32)
    o_ref[...] = jnp.tanh(t) * 0.5 + 0.5


def _tc_bilinear(q, h, bmi, bmj):
    n, d = q.shape
    return pl.pallas_call(
        _bilin_body,
        grid=(pl.cdiv(n, bmi), pl.cdiv(n, bmj)),
        in_specs=[pl.BlockSpec((bmi, d), lambda i, j: (i, 0)),
                  pl.BlockSpec((bmj, d), lambda i, j: (j, 0))],
        out_specs=pl.BlockSpec((bmi, bmj), lambda i, j: (i, j)),
        out_shape=jax.ShapeDtypeStruct((n, n), jnp.float32),
    )(q, h)


def _bcast_lane(v16, e):
    idx = jnp.full((16, 1), e, jnp.int32)
    dn = lax.GatherDimensionNumbers(
        offset_dims=(), collapsed_slice_dims=(0,), start_index_map=(0,))
    return lax.gather(v16, idx, dn, (1,),
                      mode=lax.GatherScatterMode.PROMISE_IN_BOUNDS)


def _make_spmm(np_, d, e_pad, tc_tiling=True):
    per_core = e_pad // NC
    per_tile = per_core // NS
    n_chunks = per_tile // CH
    assert n_chunks % (NB * NQ // 2) == 0
    rpt = np_ // NS
    zr = CH
    lanes = d // 16
    mesh = plsc.VectorSubcoreMesh(core_axis_name="c", subcore_axis_name="s")

    @functools.partial(
        pl.kernel,
        mesh=mesh,
        out_type=jax.ShapeDtypeStruct((NC, np_, d), jnp.float32),
        compiler_params=pltpu.CompilerParams(use_tc_tiling_on_sc=tc_tiling),
        scratch_types=[
            pltpu.VMEM((NQ, CH), jnp.int32),
            pltpu.VMEM((NQ * CH // 16, 16), jnp.int32),
            pltpu.VMEM((NQ, CH), jnp.float32),
            [pltpu.VMEM((CH, d), jnp.float32) for _ in range(NB)],
            pltpu.VMEM_SHARED((np_, d), jnp.float32),
            [pltpu.SemaphoreType.DMA for _ in range(NB)],
            [pltpu.SemaphoreType.DMA for _ in range(NQ)],
            [pltpu.SemaphoreType.DMA for _ in range(NB)],
            pltpu.SemaphoreType.DMA,
        ],
    )
    def spmm(h_hbm, cols_hbm, rows_hbm, vals_hbm, out_hbm,
             colv, rowv, valv, gath, acc, gsem, isem, ssem, sem):
        c = lax.axis_index("c")
        s = lax.axis_index("s")

        zeros16 = jnp.zeros((16,), jnp.float32)

        @pl.loop(0, zr)
        def _zero_gath(r):
            for j in range(lanes):
                gath[0][r, pl.ds(j * 16, 16)] = zeros16

        stripe = s * rpt
        zcopies = [
            pltpu.make_async_copy(
                gath[0], acc.at[pl.ds(stripe + t * zr, zr)], sem)
            for t in range(rpt // zr)
        ]
        for cp in zcopies:
            cp.start()
        for cp in zcopies:
            cp.wait()
        plsc.subcore_barrier()

        cb = (c * NS + s) * n_chunks

        ng = CH // 16

        def idx_copies(k, q):
            ci = cb + k
            return [
                pltpu.make_async_copy(cols_hbm.at[ci], colv.at[q], isem[q]),
                pltpu.make_async_copy(rows_hbm.at[pl.ds(ci * ng, ng)],
                                      rowv.at[pl.ds(q * ng, ng)], isem[q]),
                pltpu.make_async_copy(vals_hbm.at[ci], valv.at[q], isem[q]),
            ]

        def fire_idx(k, q):
            for cp in idx_copies(k, q):
                cp.start()

        def drain_idx(k, q):
            for cp in idx_copies(k, q):
                cp.wait()

        def start_gather(q, b):
            pltpu.async_copy(h_hbm.at[colv.at[q]], gath[b], gsem[b])

        def wait_gather(q, b):
            pltpu.make_async_copy(h_hbm.at[colv.at[q]], gath[b], gsem[b]).wait()

        fire_idx(0, 0)
        fire_idx(1, 1)
        drain_idx(0, 0)
        start_gather(0, 0)
        drain_idx(1, 1)
        start_gather(1, 1)

        @pl.loop(0, n_chunks, step=NQ)
        def _chunk(k):
            for q in range(NQ):
                kk = k + q
                b = q % NB
                qn = (q + 2) % NQ

                @pl.when(kk + 2 < n_chunks)
                def _fire_next_idx():
                    fire_idx(kk + 2, qn)

                wait_gather(q, b)
                g = gath[b]

                def grp_scatter(gi):
                    return pltpu.make_async_copy(
                        g.at[pl.ds(gi * 16, 16)],
                        acc.at[rowv.at[q * ng + gi]], ssem[b])

                @pl.loop(0, ng)
                def _scale(gi):
                    ebase = gi * 16
                    v16 = valv[q, pl.ds(ebase, 16)]
                    for e in range(16):
                        vv = _bcast_lane(v16, e)
                        for j in range(lanes):
                            sl = pl.ds(j * 16, 16)
                            g[ebase + e, sl] = g[ebase + e, sl] * vv
                    grp_scatter(gi).start(add=True)

                @pl.loop(0, ng)
                def _drain(gi):
                    grp_scatter(gi).wait()

                @pl.when(kk + 2 < n_chunks)
                def _issue_next():
                    drain_idx(kk + 2, qn)
                    start_gather(qn, b)

        plsc.subcore_barrier()
        ocopies = [
            pltpu.make_async_copy(
                acc.at[pl.ds(stripe + t * zr, zr)],
                out_hbm.at[c, pl.ds(stripe + t * zr, zr)], sem)
            for t in range(rpt // zr)
        ]
        for cp in ocopies:
            cp.start()
        for cp in ocopies:
            cp.wait()

    return spmm


def _spmm_partials(h, cols, rows, vals):
    d = h.shape[1]
    np_ = ((h.shape[0] + NS * 128 - 1) // (NS * 128)) * (NS * 128)
    e_pad = cols.shape[0]
    shp = (e_pad // CH, CH)
    return _make_spmm(np_, d, e_pad)(
        h, cols.reshape(shp), rows.reshape(-1, 16), vals.reshape(shp))


def kernel(x, adj_indices, adj_values, W1, W2, Wb):
    n = x.shape[0]
    e = adj_values.shape[0]
    group = NC * NS * CH * NB
    e_pad = ((e + group - 1) // group) * group

    rows = jnp.concatenate(
        [adj_indices[0], jnp.zeros((e_pad - e,), jnp.int32)])
    cols = jnp.concatenate(
        [adj_indices[1], jnp.zeros((e_pad - e,), jnp.int32)])
    vals = jnp.concatenate(
        [adj_values, jnp.zeros((e_pad - e,), jnp.float32)])

    w2p = jnp.pad(W2, ((0, 0), (0, 128 - W2.shape[1])))

    g1 = _tc_matmul(x, W1, bm=2000)
    p1 = _spmm_partials(g1, cols, rows, vals)
    g2 = _tc_elu_matmul(p1, w2p, n, bm=2000)
    p2 = _spmm_partials(g2, cols, rows, vals)
    h2, q = _tc_head(p2, Wb, n, bm=2000)
    return _tc_bilinear(q, h2, bmi=2048, bmj=2048)
